# Initial kernel scaffold; baseline (speedup 1.0000x reference)
import jax
import jax.numpy as jnp
from jax.experimental import pallas as pl


def kernel(x, edge_index, edge_attr_cat, edge_attr_num, edge_attr_time, cat_emb, num_weight, num_bias, time_weight, time_bias):
    raise NotImplementedError("write your pallas kernel here")


# speedup vs baseline: 1.7065x; 1.7065x over previous
import functools

import jax
import jax.numpy as jnp
from jax import lax
from jax.experimental import pallas as pl
from jax.experimental.pallas import tpu as pltpu
from jax.experimental.pallas import tpu_sc as plsc

_C = 128
_N_CAT = 6
_N_NUM = 4
_N_COLS = 11
_PERIODS = jnp.array(
    [60.0, 3600.0, 86400.0, 604800.0, 2592000.0, 31536000.0], dtype=jnp.float32
)
_N_PERIODS = 6


@functools.lru_cache(maxsize=None)
def _make_sc_gather(E: int, V: int):
    info = plsc.get_sparse_core_info()
    NC, NS = info.num_cores, info.num_subcores
    NW = NC * NS
    assert E % (8 * NW) == 0
    npt = E // NW

    mesh = plsc.VectorSubcoreMesh(core_axis_name="c", subcore_axis_name="s")

    @functools.partial(
        pl.kernel,
        out_type=jax.ShapeDtypeStruct((E, _N_COLS * _C), jnp.float32),
        mesh=mesh,
        scratch_types=[
            pltpu.VMEM((npt,), jnp.int32),
            pltpu.VMEM((npt, _C), jnp.float32),
            pltpu.SemaphoreType.DMA,
        ],
    )
    def sc_gather(idx_hbm, e0, e1, e2, e3, e4, e5, out_hbm, idx_v, rows_v, sem):
        embs = [e0, e1, e2, e3, e4, e5]
        wid = lax.axis_index("s") * NC + lax.axis_index("c")
        base = wid * npt
        for c in range(_N_CAT):
            pltpu.sync_copy(idx_hbm.at[c, pl.ds(base, npt)], idx_v)
            pltpu.async_copy(embs[c].at[idx_v], rows_v, sem).wait()
            pltpu.sync_copy(
                rows_v, out_hbm.at[pl.ds(base, npt), pl.ds(c * _C, _C)]
            )

    return sc_gather


def _tc_body(alias_ref, num_ref, cyc_ref, nw_ref, nb_ref, tw_ref, tb_ref, out_ref):
    del alias_ref
    j = pl.program_id(1)
    for jc in range(_N_NUM):

        @pl.when(j == jc)
        def _(jc=jc):
            out_ref[...] = (
                num_ref[:, jc : jc + 1] * nw_ref[jc : jc + 1, :]
                + nb_ref[jc : jc + 1, :]
            )

    @pl.when(j == _N_NUM)
    def _():
        acc = lax.dot_general(
            cyc_ref[...],
            tw_ref[...],
            (((1,), (0,)), ((), ())),
            preferred_element_type=jnp.float32,
        )
        out_ref[...] = acc + tb_ref[...]


@functools.lru_cache(maxsize=None)
def _make_tc_dense(E: int, B: int):
    F = 2 * _N_PERIODS
    return pl.pallas_call(
        _tc_body,
        grid=(E // B, _N_NUM + 1),
        in_specs=[
            pl.BlockSpec(memory_space=pl.ANY),
            pl.BlockSpec((B, _N_NUM), lambda i, j: (i, 0)),
            pl.BlockSpec((B, F), lambda i, j: (i, 0)),
            pl.BlockSpec((_N_NUM, _C), lambda i, j: (0, 0)),
            pl.BlockSpec((_N_NUM, _C), lambda i, j: (0, 0)),
            pl.BlockSpec((F, _C), lambda i, j: (0, 0)),
            pl.BlockSpec((1, _C), lambda i, j: (0, 0)),
        ],
        out_specs=pl.BlockSpec((B, _C), lambda i, j: (i, j + _N_CAT)),
        out_shape=jax.ShapeDtypeStruct((E, _N_COLS * _C), jnp.float32),
        input_output_aliases={0: 0},
    )


def kernel(x, edge_index, edge_attr_cat, edge_attr_num, edge_attr_time,
           cat_emb, num_weight, num_bias, time_weight, time_bias):
    del x, edge_index
    E = edge_attr_cat.shape[0]
    V = cat_emb.shape[1]

    t = edge_attr_time.astype(jnp.float32)
    ang = 2.0 * jnp.pi * t[:, :, None] / _PERIODS[None, None, :]
    cyc = jnp.concatenate([jnp.sin(ang), jnp.cos(ang)], axis=-1)
    cyc = cyc.reshape(E, 2 * _N_PERIODS)

    idx_t = edge_attr_cat.T
    embs = [cat_emb[c] for c in range(_N_CAT)]

    out2d = _make_sc_gather(E, V)(idx_t, *embs)
    out2d = _make_tc_dense(E, 2048)(
        out2d,
        edge_attr_num,
        cyc,
        num_weight,
        num_bias,
        time_weight.reshape(2 * _N_PERIODS, _C),
        time_bias,
    )
    return out2d.reshape(E, _N_COLS, _C)


# speedup vs baseline: 4.0739x; 2.3873x over previous
import functools

import jax
import jax.numpy as jnp
from jax import lax
from jax.experimental import pallas as pl
from jax.experimental.pallas import tpu as pltpu
from jax.experimental.pallas import tpu_sc as plsc

_C = 128
_N_CAT = 6
_N_NUM = 4
_N_COLS = 11
_PERIODS = jnp.array(
    [60.0, 3600.0, 86400.0, 604800.0, 2592000.0, 31536000.0], dtype=jnp.float32
)
_N_PERIODS = 6
_F = 2 * _N_PERIODS + _N_NUM


@functools.lru_cache(maxsize=None)
def _make_sc_gather(E: int, V: int):
    info = plsc.get_sparse_core_info()
    NC, NS = info.num_cores, info.num_subcores
    NW = NC * NS
    assert E % (8 * NW) == 0
    npt = E // NW

    mesh = plsc.VectorSubcoreMesh(core_axis_name="c", subcore_axis_name="s")

    @functools.partial(
        pl.kernel,
        out_type=jax.ShapeDtypeStruct((_N_COLS, E, _C), jnp.float32),
        mesh=mesh,
        scratch_types=[
            pltpu.VMEM((npt,), jnp.int32),
            pltpu.VMEM((npt, _C), jnp.float32),
            pltpu.SemaphoreType.DMA,
        ],
    )
    def sc_gather(idx_hbm, e0, e1, e2, e3, e4, e5, out_hbm, idx_v, rows_v, sem):
        embs = [e0, e1, e2, e3, e4, e5]
        wid = lax.axis_index("s") * NC + lax.axis_index("c")
        base = wid * npt
        for c in range(_N_CAT):
            pltpu.sync_copy(idx_hbm.at[c, pl.ds(base, npt)], idx_v)
            pltpu.async_copy(embs[c].at[idx_v], rows_v, sem).wait()
            pltpu.sync_copy(rows_v, out_hbm.at[c, pl.ds(base, npt)])

    return sc_gather


def _tc_body(alias_ref, f_ref, w_ref, b_ref, out_ref):
    del alias_ref
    acc = lax.dot_general(
        f_ref[...],
        w_ref[0],
        (((0,), (0,)), ((), ())),
        preferred_element_type=jnp.float32,
    )
    out_ref[0] = acc + b_ref[0]


@functools.lru_cache(maxsize=None)
def _make_tc_dense(E: int, B: int):
    return pl.pallas_call(
        _tc_body,
        grid=(E // B, _N_COLS - _N_CAT),
        in_specs=[
            pl.BlockSpec(memory_space=pl.ANY),
            pl.BlockSpec((_F, B), lambda i, j: (0, i)),
            pl.BlockSpec((1, _F, _C), lambda i, j: (j, 0, 0)),
            pl.BlockSpec((1, 1, _C), lambda i, j: (j, 0, 0)),
        ],
        out_specs=pl.BlockSpec((1, B, _C), lambda i, j: (j + _N_CAT, i, 0)),
        out_shape=jax.ShapeDtypeStruct((_N_COLS, E, _C), jnp.float32),
        input_output_aliases={0: 0},
    )


def kernel(x, edge_index, edge_attr_cat, edge_attr_num, edge_attr_time,
           cat_emb, num_weight, num_bias, time_weight, time_bias):
    del x, edge_index
    E = edge_attr_cat.shape[0]
    V = cat_emb.shape[1]
    NT = _N_COLS - _N_CAT

    t_row = edge_attr_time.astype(jnp.float32).T
    ang = 2.0 * jnp.pi * t_row / _PERIODS[:, None]
    cyc = jnp.concatenate([jnp.sin(ang), jnp.cos(ang)], axis=0)
    feats = jnp.concatenate([cyc, edge_attr_num.T], axis=0)

    tw2 = time_weight.reshape(2 * _N_PERIODS, _C)
    w_all = jnp.zeros((NT, _F, _C), jnp.float32)
    for j in range(_N_NUM):
        w_all = w_all.at[j, 2 * _N_PERIODS + j].set(num_weight[j])
    w_all = w_all.at[_N_NUM, : 2 * _N_PERIODS].set(tw2)
    b_all = jnp.concatenate([num_bias, time_bias], axis=0)[:, None, :]

    idx_t = edge_attr_cat.T
    embs = [cat_emb[c] for c in range(_N_CAT)]

    out = _make_sc_gather(E, V)(idx_t, *embs)
    out = _make_tc_dense(E, 2048)(out, feats, w_all, b_all)
    return jnp.transpose(out, (1, 0, 2))


# speedup vs baseline: 4.1056x; 1.0078x over previous
import functools

import jax
import jax.numpy as jnp
from jax import lax
from jax.experimental import pallas as pl
from jax.experimental.pallas import tpu as pltpu
from jax.experimental.pallas import tpu_sc as plsc

_C = 128
_N_CAT = 6
_N_NUM = 4
_N_COLS = 11
_PERIODS = jnp.array(
    [60.0, 3600.0, 86400.0, 604800.0, 2592000.0, 31536000.0], dtype=jnp.float32
)
_N_PERIODS = 6
_F = 2 * _N_PERIODS + _N_NUM


@functools.lru_cache(maxsize=None)
def _make_sc_gather(E: int, V: int):
    info = plsc.get_sparse_core_info()
    NC, NS = info.num_cores, info.num_subcores
    NW = NC * NS
    assert E % (8 * NW) == 0
    npt = E // NW

    mesh = plsc.VectorSubcoreMesh(core_axis_name="c", subcore_axis_name="s")

    CH = 128
    SPLIT = npt // CH
    NS_TEPS = _N_CAT * SPLIT

    @functools.partial(
        pl.kernel,
        out_type=jax.ShapeDtypeStruct((_N_COLS, E, _C), jnp.float32),
        mesh=mesh,
        scratch_types=[
            pltpu.VMEM((NS_TEPS, CH), jnp.int32),
            pltpu.VMEM((CH, _C), jnp.float32),
            pltpu.VMEM((CH, _C), jnp.float32),
            pltpu.SemaphoreType.DMA,
            pltpu.SemaphoreType.DMA,
            pltpu.SemaphoreType.DMA,
            pltpu.SemaphoreType.DMA,
        ],
    )
    def sc_gather(idx_hbm, e0, e1, e2, e3, e4, e5, out_hbm,
                  idx_v, buf0, buf1, g0, g1, w0, w1):
        embs = [e0, e1, e2, e3, e4, e5]
        bufs = [buf0, buf1]
        gsems = [g0, g1]
        wsems = [w0, w1]
        wid = lax.axis_index("s") * NC + lax.axis_index("c")
        base = wid * npt

        pltpu.sync_copy(idx_hbm.at[wid], idx_v)

        def chunk(s):
            c, h = divmod(s, SPLIT)
            off = h * CH
            src = embs[c].at[idx_v.at[s]]
            dst = out_hbm.at[c, pl.ds(base + off, CH)]
            return src, dst

        gh = [None] * NS_TEPS
        wh = [None] * NS_TEPS
        for s in range(NS_TEPS):
            b = s % 2
            if s >= 2:
                wh[s - 2].wait()
            src, _ = chunk(s)
            gh[s] = pltpu.async_copy(src, bufs[b], gsems[b])
            if s >= 1:
                pb = (s - 1) % 2
                gh[s - 1].wait()
                _, dst = chunk(s - 1)
                wh[s - 1] = pltpu.async_copy(bufs[pb], dst, wsems[pb])
        last = NS_TEPS - 1
        gh[last].wait()
        _, dst = chunk(last)
        wh[last] = pltpu.async_copy(bufs[last % 2], dst, wsems[last % 2])
        wh[last - 1].wait()
        wh[last].wait()

    return sc_gather


def _tc_body(alias_ref, f_ref, w_ref, b_ref, out_ref):
    del alias_ref
    acc = lax.dot_general(
        f_ref[...],
        w_ref[0],
        (((0,), (0,)), ((), ())),
        preferred_element_type=jnp.float32,
    )
    out_ref[0] = acc + b_ref[0]


@functools.lru_cache(maxsize=None)
def _make_tc_dense(E: int, B: int):
    return pl.pallas_call(
        _tc_body,
        grid=(E // B, _N_COLS - _N_CAT),
        in_specs=[
            pl.BlockSpec(memory_space=pl.ANY),
            pl.BlockSpec((_F, B), lambda i, j: (0, i)),
            pl.BlockSpec((1, _F, _C), lambda i, j: (j, 0, 0)),
            pl.BlockSpec((1, 1, _C), lambda i, j: (j, 0, 0)),
        ],
        out_specs=pl.BlockSpec((1, B, _C), lambda i, j: (j + _N_CAT, i, 0)),
        out_shape=jax.ShapeDtypeStruct((_N_COLS, E, _C), jnp.float32),
        input_output_aliases={0: 0},
    )


def kernel(x, edge_index, edge_attr_cat, edge_attr_num, edge_attr_time,
           cat_emb, num_weight, num_bias, time_weight, time_bias):
    del x, edge_index
    E = edge_attr_cat.shape[0]
    V = cat_emb.shape[1]
    NT = _N_COLS - _N_CAT

    t_row = edge_attr_time.astype(jnp.float32).T
    ang = 2.0 * jnp.pi * t_row / _PERIODS[:, None]
    cyc = jnp.concatenate([jnp.sin(ang), jnp.cos(ang)], axis=0)
    feats = jnp.concatenate([cyc, edge_attr_num.T], axis=0)

    tw2 = time_weight.reshape(2 * _N_PERIODS, _C)
    w_all = jnp.zeros((NT, _F, _C), jnp.float32)
    for j in range(_N_NUM):
        w_all = w_all.at[j, 2 * _N_PERIODS + j].set(num_weight[j])
    w_all = w_all.at[_N_NUM, : 2 * _N_PERIODS].set(tw2)
    b_all = jnp.concatenate([num_bias, time_bias], axis=0)[:, None, :]

    info = plsc.get_sparse_core_info()
    NW = info.num_cores * info.num_subcores
    npt = E // NW
    idx_g = (
        edge_attr_cat.T.reshape(_N_CAT, NW, npt // 128, 128)
        .transpose(1, 0, 2, 3)
        .reshape(NW, _N_CAT * (npt // 128), 128)
    )
    embs = [cat_emb[c] for c in range(_N_CAT)]

    out = _make_sc_gather(E, V)(idx_g, *embs)
    out = _make_tc_dense(E, 2048)(out, feats, w_all, b_all)
    return jnp.transpose(out, (1, 0, 2))


# speedup vs baseline: 5.7067x; 1.3900x over previous
import functools

import jax
import jax.numpy as jnp
from jax import lax
from jax.experimental import pallas as pl
from jax.experimental.pallas import tpu as pltpu
from jax.experimental.pallas import tpu_sc as plsc

_C = 128
_N_CAT = 6
_N_NUM = 4
_N_COLS = 11
_PERIODS = jnp.array(
    [60.0, 3600.0, 86400.0, 604800.0, 2592000.0, 31536000.0], dtype=jnp.float32
)
_N_PERIODS = 6
_F = 2 * _N_PERIODS + _N_NUM


@functools.lru_cache(maxsize=None)
def _make_sc_gather(E: int, V: int):
    info = plsc.get_sparse_core_info()
    NC, NS = info.num_cores, info.num_subcores
    NW = NC * NS
    assert E % (8 * NW) == 0
    npt = E // NW

    mesh = plsc.VectorSubcoreMesh(core_axis_name="c", subcore_axis_name="s")

    CH = 128
    SPLIT = npt // CH
    NS_TEPS = _N_CAT * SPLIT

    @functools.partial(
        pl.kernel,
        out_type=jax.ShapeDtypeStruct((_N_COLS, E, _C), jnp.float32),
        mesh=mesh,
        scratch_types=[
            pltpu.VMEM((NS_TEPS, CH), jnp.int32),
            pltpu.VMEM((CH, _C), jnp.float32),
            pltpu.VMEM((CH, _C), jnp.float32),
            pltpu.VMEM_SHARED((_N_CAT * V, _C), jnp.float32),
            pltpu.SemaphoreType.DMA,
            pltpu.SemaphoreType.DMA,
            pltpu.SemaphoreType.DMA,
            pltpu.SemaphoreType.DMA,
        ],
    )
    def sc_gather(idx_hbm, emb_hbm, out_hbm,
                  idx_v, buf0, buf1, spm, g0, g1, w0, w1):
        bufs = [buf0, buf1]
        gsems = [g0, g1]
        wsems = [w0, w1]
        sid = lax.axis_index("s")
        wid = sid * NC + lax.axis_index("c")
        base = wid * npt

        @pl.when(sid == 0)
        def _():
            pltpu.sync_copy(emb_hbm, spm)

        pltpu.sync_copy(idx_hbm.at[wid], idx_v)
        plsc.subcore_barrier()

        def chunk(s):
            c, h = divmod(s, SPLIT)
            off = h * CH
            src = spm.at[idx_v.at[s]]
            dst = out_hbm.at[c, pl.ds(base + off, CH)]
            return src, dst

        gh = [None] * NS_TEPS
        wh = [None] * NS_TEPS
        for s in range(NS_TEPS):
            b = s % 2
            if s >= 2:
                wh[s - 2].wait()
            src, _ = chunk(s)
            gh[s] = pltpu.async_copy(src, bufs[b], gsems[b])
            if s >= 1:
                pb = (s - 1) % 2
                gh[s - 1].wait()
                _, dst = chunk(s - 1)
                wh[s - 1] = pltpu.async_copy(bufs[pb], dst, wsems[pb])
        last = NS_TEPS - 1
        gh[last].wait()
        _, dst = chunk(last)
        wh[last] = pltpu.async_copy(bufs[last % 2], dst, wsems[last % 2])
        wh[last - 1].wait()
        wh[last].wait()

    return sc_gather


def _tc_body(alias_ref, f_ref, w_ref, b_ref, out_ref):
    del alias_ref
    acc = lax.dot_general(
        f_ref[...],
        w_ref[0],
        (((0,), (0,)), ((), ())),
        preferred_element_type=jnp.float32,
    )
    out_ref[0] = acc + b_ref[0]


@functools.lru_cache(maxsize=None)
def _make_tc_dense(E: int, B: int):
    return pl.pallas_call(
        _tc_body,
        grid=(E // B, _N_COLS - _N_CAT),
        in_specs=[
            pl.BlockSpec(memory_space=pl.ANY),
            pl.BlockSpec((_F, B), lambda i, j: (0, i)),
            pl.BlockSpec((1, _F, _C), lambda i, j: (j, 0, 0)),
            pl.BlockSpec((1, 1, _C), lambda i, j: (j, 0, 0)),
        ],
        out_specs=pl.BlockSpec((1, B, _C), lambda i, j: (j + _N_CAT, i, 0)),
        out_shape=jax.ShapeDtypeStruct((_N_COLS, E, _C), jnp.float32),
        input_output_aliases={0: 0},
    )


def kernel(x, edge_index, edge_attr_cat, edge_attr_num, edge_attr_time,
           cat_emb, num_weight, num_bias, time_weight, time_bias):
    del x, edge_index
    E = edge_attr_cat.shape[0]
    V = cat_emb.shape[1]
    NT = _N_COLS - _N_CAT

    t_row = edge_attr_time.astype(jnp.float32).T
    ang = 2.0 * jnp.pi * t_row / _PERIODS[:, None]
    cyc = jnp.concatenate([jnp.sin(ang), jnp.cos(ang)], axis=0)
    feats = jnp.concatenate([cyc, edge_attr_num.T], axis=0)

    tw2 = time_weight.reshape(2 * _N_PERIODS, _C)
    w_all = jnp.zeros((NT, _F, _C), jnp.float32)
    for j in range(_N_NUM):
        w_all = w_all.at[j, 2 * _N_PERIODS + j].set(num_weight[j])
    w_all = w_all.at[_N_NUM, : 2 * _N_PERIODS].set(tw2)
    b_all = jnp.concatenate([num_bias, time_bias], axis=0)[:, None, :]

    info = plsc.get_sparse_core_info()
    NW = info.num_cores * info.num_subcores
    npt = E // NW
    gidx = edge_attr_cat.T + (jnp.arange(_N_CAT, dtype=jnp.int32) * V)[:, None]
    idx_g = (
        gidx.reshape(_N_CAT, NW, npt // 128, 128)
        .transpose(1, 0, 2, 3)
        .reshape(NW, _N_CAT * (npt // 128), 128)
    )
    emb_all = cat_emb.reshape(_N_CAT * V, _C)

    out = _make_sc_gather(E, V)(idx_g, emb_all)
    out = _make_tc_dense(E, 2048)(out, feats, w_all, b_all)
    return jnp.transpose(out, (1, 0, 2))


# speedup vs baseline: 6.4893x; 1.1371x over previous
import functools

import jax
import jax.numpy as jnp
from jax import lax
from jax.experimental import pallas as pl
from jax.experimental.pallas import tpu as pltpu
from jax.experimental.pallas import tpu_sc as plsc

_C = 128
_N_CAT = 6
_N_NUM = 4
_N_COLS = 11
_PERIODS = jnp.array(
    [60.0, 3600.0, 86400.0, 604800.0, 2592000.0, 31536000.0], dtype=jnp.float32
)
_N_PERIODS = 6
_F = 2 * _N_PERIODS + _N_NUM


@functools.lru_cache(maxsize=None)
def _make_sc_gather(E: int, V: int):
    info = plsc.get_sparse_core_info()
    NC, NS = info.num_cores, info.num_subcores
    NW = NC * NS
    assert E % (8 * NW) == 0
    npt = E // NW

    mesh = plsc.VectorSubcoreMesh(core_axis_name="c", subcore_axis_name="s")

    CH = 128
    SPLIT = npt // CH
    NS_TEPS = _N_CAT * SPLIT

    @functools.partial(
        pl.kernel,
        out_type=jax.ShapeDtypeStruct((_N_COLS, E, _C), jnp.float32),
        mesh=mesh,
        scratch_types=[
            pltpu.VMEM((NS_TEPS, CH), jnp.int32),
            pltpu.VMEM((CH, _C), jnp.float32),
            pltpu.VMEM((CH, _C), jnp.float32),
            pltpu.VMEM_SHARED((_N_CAT * V, _C), jnp.float32),
            pltpu.SemaphoreType.DMA,
            pltpu.SemaphoreType.DMA,
            pltpu.SemaphoreType.DMA,
            pltpu.SemaphoreType.DMA,
        ],
    )
    def sc_gather(idx_hbm, emb_hbm, out_hbm,
                  idx_v, buf0, buf1, spm, g0, g1, w0, w1):
        bufs = [buf0, buf1]
        gsems = [g0, g1]
        wsems = [w0, w1]
        sid = lax.axis_index("s")
        wid = sid * NC + lax.axis_index("c")
        base = wid * npt

        @pl.when(sid == 0)
        def _():
            pltpu.sync_copy(emb_hbm, spm)

        pltpu.sync_copy(idx_hbm.at[wid], idx_v)
        plsc.subcore_barrier()

        def chunk(s):
            c, h = divmod(s, SPLIT)
            off = h * CH
            src = spm.at[idx_v.at[s]]
            dst = out_hbm.at[c, pl.ds(base + off, CH)]
            return src, dst

        gh = [None] * NS_TEPS
        wh = [None] * NS_TEPS
        for s in range(NS_TEPS):
            b = s % 2
            if s >= 2:
                wh[s - 2].wait()
            src, _ = chunk(s)
            gh[s] = pltpu.async_copy(src, bufs[b], gsems[b])
            if s >= 1:
                pb = (s - 1) % 2
                gh[s - 1].wait()
                _, dst = chunk(s - 1)
                wh[s - 1] = pltpu.async_copy(bufs[pb], dst, wsems[pb])
        last = NS_TEPS - 1
        gh[last].wait()
        _, dst = chunk(last)
        wh[last] = pltpu.async_copy(bufs[last % 2], dst, wsems[last % 2])
        wh[last - 1].wait()
        wh[last].wait()

    return sc_gather


def _tc_body(alias_ref, f_ref, w_ref, b_ref, out_ref):
    del alias_ref
    acc = lax.dot_general(
        f_ref[...],
        w_ref[0],
        (((0,), (0,)), ((), ())),
        preferred_element_type=jnp.float32,
    )
    out_ref[0] = acc + b_ref[0]


@functools.lru_cache(maxsize=None)
def _make_tc_dense(E: int, B: int):
    return pl.pallas_call(
        _tc_body,
        grid=(E // B, _N_COLS - _N_CAT),
        in_specs=[
            pl.BlockSpec(memory_space=pl.ANY),
            pl.BlockSpec((_F, B), lambda i, j: (0, i)),
            pl.BlockSpec((1, _F, _C), lambda i, j: (j, 0, 0)),
            pl.BlockSpec((1, 1, _C), lambda i, j: (j, 0, 0)),
        ],
        out_specs=pl.BlockSpec((1, B, _C), lambda i, j: (j + _N_CAT, i, 0)),
        out_shape=jax.ShapeDtypeStruct((_N_COLS, E, _C), jnp.float32),
        input_output_aliases={0: 0},
    )


def kernel(x, edge_index, edge_attr_cat, edge_attr_num, edge_attr_time,
           cat_emb, num_weight, num_bias, time_weight, time_bias):
    del x, edge_index
    E = edge_attr_cat.shape[0]
    V = cat_emb.shape[1]
    NT = _N_COLS - _N_CAT

    t_row = edge_attr_time.astype(jnp.float32).T
    ang = 2.0 * jnp.pi * t_row / _PERIODS[:, None]
    cyc = jnp.concatenate([jnp.sin(ang), jnp.cos(ang)], axis=0)
    feats = jnp.concatenate([cyc, edge_attr_num.T], axis=0)

    tw2 = time_weight.reshape(2 * _N_PERIODS, _C)
    w_all = jnp.zeros((NT, _F, _C), jnp.float32)
    for j in range(_N_NUM):
        w_all = w_all.at[j, 2 * _N_PERIODS + j].set(num_weight[j])
    w_all = w_all.at[_N_NUM, : 2 * _N_PERIODS].set(tw2)
    b_all = jnp.concatenate([num_bias, time_bias], axis=0)[:, None, :]

    info = plsc.get_sparse_core_info()
    NW = info.num_cores * info.num_subcores
    npt = E // NW
    gidx = edge_attr_cat.T + (jnp.arange(_N_CAT, dtype=jnp.int32) * V)[:, None]
    idx_g = (
        gidx.reshape(_N_CAT, NW, npt // 128, 128)
        .transpose(1, 0, 2, 3)
        .reshape(NW, _N_CAT * (npt // 128), 128)
    )
    emb_all = cat_emb.reshape(_N_CAT * V, _C)

    out = _make_sc_gather(E, V)(idx_g, emb_all)
    out = _make_tc_dense(E, 4096)(out, feats, w_all, b_all)
    return jnp.transpose(out, (1, 0, 2))


# speedup vs baseline: 6.9836x; 1.0762x over previous
import functools

import jax
import jax.numpy as jnp
from jax import lax
from jax.experimental import pallas as pl
from jax.experimental.pallas import tpu as pltpu
from jax.experimental.pallas import tpu_sc as plsc

_C = 128
_N_CAT = 6
_N_NUM = 4
_N_COLS = 11
_PERIODS = jnp.array(
    [60.0, 3600.0, 86400.0, 604800.0, 2592000.0, 31536000.0], dtype=jnp.float32
)
_N_PERIODS = 6
_F = 2 * _N_PERIODS + _N_NUM


@functools.lru_cache(maxsize=None)
def _make_sc_gather(E: int, V: int):
    info = plsc.get_sparse_core_info()
    NC, NS = info.num_cores, info.num_subcores
    NW = NC * NS
    assert E % (8 * NW) == 0
    npt = E // NW

    mesh = plsc.VectorSubcoreMesh(core_axis_name="c", subcore_axis_name="s")

    CH = 128
    SPLIT = npt // CH
    NS_TEPS = _N_CAT * SPLIT

    @functools.partial(
        pl.kernel,
        out_type=jax.ShapeDtypeStruct((_N_COLS, E, _C), jnp.float32),
        mesh=mesh,
        scratch_types=[
            pltpu.VMEM((NS_TEPS, CH), jnp.int32),
            pltpu.VMEM((CH, _C), jnp.float32),
            pltpu.VMEM((CH, _C), jnp.float32),
            pltpu.VMEM_SHARED((_N_CAT * V, _C), jnp.float32),
            pltpu.SemaphoreType.DMA,
            pltpu.SemaphoreType.DMA,
            pltpu.SemaphoreType.DMA,
            pltpu.SemaphoreType.DMA,
        ],
    )
    def sc_gather(idx_hbm, emb_hbm, out_hbm,
                  idx_v, buf0, buf1, spm, g0, g1, w0, w1):
        bufs = [buf0, buf1]
        gsems = [g0, g1]
        wsems = [w0, w1]
        sid = lax.axis_index("s")
        wid = sid * NC + lax.axis_index("c")
        base = wid * npt

        @pl.when(sid == 0)
        def _():
            pltpu.sync_copy(emb_hbm, spm)

        pltpu.sync_copy(idx_hbm.at[wid], idx_v)
        plsc.subcore_barrier()

        def chunk(s):
            c, h = divmod(s, SPLIT)
            off = h * CH
            src = spm.at[idx_v.at[s]]
            dst = out_hbm.at[c, pl.ds(base + off, CH)]
            return src, dst

        gh = [None] * NS_TEPS
        wh = [None] * NS_TEPS
        for s in range(NS_TEPS):
            b = s % 2
            if s >= 2:
                wh[s - 2].wait()
            src, _ = chunk(s)
            gh[s] = pltpu.async_copy(src, bufs[b], gsems[b])
            if s >= 1:
                pb = (s - 1) % 2
                gh[s - 1].wait()
                _, dst = chunk(s - 1)
                wh[s - 1] = pltpu.async_copy(bufs[pb], dst, wsems[pb])
        last = NS_TEPS - 1
        gh[last].wait()
        _, dst = chunk(last)
        wh[last] = pltpu.async_copy(bufs[last % 2], dst, wsems[last % 2])
        wh[last - 1].wait()
        wh[last].wait()

    return sc_gather


def _tc_body(alias_ref, f_ref, w_ref, b_ref, out_ref):
    del alias_ref
    acc = lax.dot_general(
        f_ref[...],
        w_ref[0],
        (((0,), (0,)), ((), ())),
        preferred_element_type=jnp.float32,
    )
    out_ref[0] = acc + b_ref[0]


@functools.lru_cache(maxsize=None)
def _make_tc_dense(E: int, B: int):
    return pl.pallas_call(
        _tc_body,
        grid=(E // B, _N_COLS - _N_CAT),
        in_specs=[
            pl.BlockSpec(memory_space=pl.ANY),
            pl.BlockSpec((_F, B), lambda i, j: (0, i)),
            pl.BlockSpec((1, _F, _C), lambda i, j: (j, 0, 0)),
            pl.BlockSpec((1, 1, _C), lambda i, j: (j, 0, 0)),
        ],
        out_specs=pl.BlockSpec((1, B, _C), lambda i, j: (j + _N_CAT, i, 0)),
        out_shape=jax.ShapeDtypeStruct((_N_COLS, E, _C), jnp.float32),
        input_output_aliases={0: 0},
    )


def kernel(x, edge_index, edge_attr_cat, edge_attr_num, edge_attr_time,
           cat_emb, num_weight, num_bias, time_weight, time_bias):
    del x, edge_index
    E = edge_attr_cat.shape[0]
    V = cat_emb.shape[1]
    NT = _N_COLS - _N_CAT

    t_row = edge_attr_time.astype(jnp.float32).T
    ang = 2.0 * jnp.pi * t_row / _PERIODS[:, None]
    cyc = jnp.concatenate([jnp.sin(ang), jnp.cos(ang)], axis=0)
    feats = jnp.concatenate([cyc, edge_attr_num.T], axis=0)

    tw2 = time_weight.reshape(2 * _N_PERIODS, _C)
    w_all = jnp.zeros((NT, _F, _C), jnp.float32)
    for j in range(_N_NUM):
        w_all = w_all.at[j, 2 * _N_PERIODS + j].set(num_weight[j])
    w_all = w_all.at[_N_NUM, : 2 * _N_PERIODS].set(tw2)
    b_all = jnp.concatenate([num_bias, time_bias], axis=0)[:, None, :]

    info = plsc.get_sparse_core_info()
    NW = info.num_cores * info.num_subcores
    npt = E // NW
    gidx = edge_attr_cat.T + (jnp.arange(_N_CAT, dtype=jnp.int32) * V)[:, None]
    idx_g = (
        gidx.reshape(_N_CAT, NW, npt // 128, 128)
        .transpose(1, 0, 2, 3)
        .reshape(NW, _N_CAT * (npt // 128), 128)
    )
    emb_all = cat_emb.reshape(_N_CAT * V, _C)

    out = _make_sc_gather(E, V)(idx_g, emb_all)
    out = _make_tc_dense(E, 8192)(out, feats, w_all, b_all)
    return jnp.transpose(out, (1, 0, 2))


# speedup vs baseline: 7.0882x; 1.0150x over previous
import functools

import jax
import jax.numpy as jnp
from jax import lax
from jax.experimental import pallas as pl
from jax.experimental.pallas import tpu as pltpu
from jax.experimental.pallas import tpu_sc as plsc

_C = 128
_N_CAT = 6
_N_NUM = 4
_N_COLS = 11
_PERIODS = jnp.array(
    [60.0, 3600.0, 86400.0, 604800.0, 2592000.0, 31536000.0], dtype=jnp.float32
)
_N_PERIODS = 6
_F = 2 * _N_PERIODS + _N_NUM


@functools.lru_cache(maxsize=None)
def _make_sc_gather(E: int, V: int):
    info = plsc.get_sparse_core_info()
    NC, NS = info.num_cores, info.num_subcores
    NW = NC * NS
    assert E % (8 * NW) == 0
    npt = E // NW

    mesh = plsc.VectorSubcoreMesh(core_axis_name="c", subcore_axis_name="s")

    CH = 128
    SPLIT = npt // CH
    NS_TEPS = _N_CAT * SPLIT

    @functools.partial(
        pl.kernel,
        out_type=jax.ShapeDtypeStruct((_N_COLS, E, _C), jnp.float32),
        mesh=mesh,
        scratch_types=[
            pltpu.VMEM((NS_TEPS, CH), jnp.int32),
            pltpu.VMEM((CH, _C), jnp.float32),
            pltpu.VMEM((CH, _C), jnp.float32),
            pltpu.VMEM_SHARED((_N_CAT * V, _C), jnp.float32),
            pltpu.SemaphoreType.DMA,
            pltpu.SemaphoreType.DMA,
            pltpu.SemaphoreType.DMA,
            pltpu.SemaphoreType.DMA,
        ],
    )
    def sc_gather(idx_hbm, emb_hbm, out_hbm,
                  idx_v, buf0, buf1, spm, g0, g1, w0, w1):
        bufs = [buf0, buf1]
        gsems = [g0, g1]
        wsems = [w0, w1]
        sid = lax.axis_index("s")
        wid = sid * NC + lax.axis_index("c")
        base = wid * npt

        @pl.when(sid == 0)
        def _():
            pltpu.sync_copy(emb_hbm, spm)

        pltpu.sync_copy(idx_hbm.at[wid], idx_v)
        plsc.subcore_barrier()

        def chunk(s):
            c, h = divmod(s, SPLIT)
            off = h * CH
            src = spm.at[idx_v.at[s]]
            dst = out_hbm.at[c, pl.ds(base + off, CH)]
            return src, dst

        gh = [None] * NS_TEPS
        wh = [None] * NS_TEPS
        for s in range(NS_TEPS):
            b = s % 2
            if s >= 2:
                wh[s - 2].wait()
            src, _ = chunk(s)
            gh[s] = pltpu.async_copy(src, bufs[b], gsems[b])
            if s >= 1:
                pb = (s - 1) % 2
                gh[s - 1].wait()
                _, dst = chunk(s - 1)
                wh[s - 1] = pltpu.async_copy(bufs[pb], dst, wsems[pb])
        last = NS_TEPS - 1
        gh[last].wait()
        _, dst = chunk(last)
        wh[last] = pltpu.async_copy(bufs[last % 2], dst, wsems[last % 2])
        wh[last - 1].wait()
        wh[last].wait()

    return sc_gather


def _tc_body(alias_ref, f_ref, w_ref, b_ref, out_ref):
    del alias_ref
    acc = lax.dot_general(
        f_ref[...],
        w_ref[0],
        (((0,), (0,)), ((), ())),
        preferred_element_type=jnp.float32,
    )
    out_ref[0] = acc + b_ref[0]


@functools.lru_cache(maxsize=None)
def _make_tc_dense(E: int, B: int):
    return pl.pallas_call(
        _tc_body,
        grid=(E // B, _N_COLS - _N_CAT),
        in_specs=[
            pl.BlockSpec(memory_space=pl.ANY),
            pl.BlockSpec((_F, B), lambda i, j: (0, i)),
            pl.BlockSpec((1, _F, _C), lambda i, j: (j, 0, 0)),
            pl.BlockSpec((1, 1, _C), lambda i, j: (j, 0, 0)),
        ],
        out_specs=pl.BlockSpec((1, B, _C), lambda i, j: (j + _N_CAT, i, 0)),
        out_shape=jax.ShapeDtypeStruct((_N_COLS, E, _C), jnp.float32),
        input_output_aliases={0: 0},
    )


def kernel(x, edge_index, edge_attr_cat, edge_attr_num, edge_attr_time,
           cat_emb, num_weight, num_bias, time_weight, time_bias):
    del x, edge_index
    E = edge_attr_cat.shape[0]
    V = cat_emb.shape[1]
    NT = _N_COLS - _N_CAT

    t_row = edge_attr_time.astype(jnp.float32).T
    ang = 2.0 * jnp.pi * t_row / _PERIODS[:, None]
    cyc = jnp.concatenate([jnp.sin(ang), jnp.cos(ang)], axis=0)
    feats = jnp.concatenate([cyc, edge_attr_num.T], axis=0)

    tw2 = time_weight.reshape(2 * _N_PERIODS, _C)
    w_all = jnp.zeros((NT, _F, _C), jnp.float32)
    for j in range(_N_NUM):
        w_all = w_all.at[j, 2 * _N_PERIODS + j].set(num_weight[j])
    w_all = w_all.at[_N_NUM, : 2 * _N_PERIODS].set(tw2)
    b_all = jnp.concatenate([num_bias, time_bias], axis=0)[:, None, :]

    info = plsc.get_sparse_core_info()
    NW = info.num_cores * info.num_subcores
    npt = E // NW
    gidx = edge_attr_cat.T + (jnp.arange(_N_CAT, dtype=jnp.int32) * V)[:, None]
    idx_g = (
        gidx.reshape(_N_CAT, NW, npt // 128, 128)
        .transpose(1, 0, 2, 3)
        .reshape(NW, _N_CAT * (npt // 128), 128)
    )
    emb_all = cat_emb.reshape(_N_CAT * V, _C)

    out = _make_sc_gather(E, V)(idx_g, emb_all)
    out = _make_tc_dense(E, 16384)(out, feats, w_all, b_all)
    return jnp.transpose(out, (1, 0, 2))
